# baseline (speedup 1.0000x reference)
import functools
import jax
import jax.numpy as jnp
from jax import lax
from jax.experimental import pallas as pl
from jax.experimental.pallas import tpu as pltpu
from jax.experimental.pallas import tpu_sc as plsc

_PC_MIN = (0.0, -40.0, -3.0)
_PILLAR = 0.32
_NX = 220
_NY = 250
_NSEG = _NX * _NY
_NSEG_PAD = 56320
_N = 200000
_N_PAD = 204800
_BLK = 4096
_ZC = 2.0


def _point_stage_body(pts_ref, w_ref, gamma_ref, pid_ref, rx_ref, ry_ref,
                      rz_ref, s_ref):
    i = pl.program_id(0)
    pts = pts_ref[...]
    relx = pts[:, 0:1] - _PC_MIN[0]
    rely = pts[:, 1:2] - _PC_MIN[1]
    relz = pts[:, 2:3] - _PC_MIN[2]
    ix = jnp.clip(jnp.floor(relx / _PILLAR).astype(jnp.int32), 0, _NX - 1)
    iy = jnp.clip(jnp.floor(rely / _PILLAR).astype(jnp.int32), 0, _NY - 1)
    pid = iy * _NX + ix
    gidx = i * _BLK + jax.lax.broadcasted_iota(jnp.int32, (_BLK, 1), 0)
    valid = gidx < _N
    pid = jnp.where(valid, pid, _NSEG_PAD - 1)
    pid_ref[...] = pid[:, 0]
    rx_ref[...] = relx[:, 0]
    ry_ref[...] = rely[:, 0]
    rz_ref[...] = relz[:, 0]
    wf = w_ref[...] * gamma_ref[...][None, :]
    rel = jnp.concatenate([relx, rely, relz], axis=1)
    u = jnp.concatenate([pts, rel, rel], axis=1)
    s = u[:, 0:1] * wf[0][None, :]
    for k in range(1, 10):
        s = s + u[:, k:k + 1] * wf[k][None, :]
    s_ref[...] = jnp.concatenate(
        [s, jnp.zeros((_BLK, 64), jnp.float32)], axis=1)


def _point_stage(points_pad, w, gamma):
    grid = _N_PAD // _BLK
    out1d = jax.ShapeDtypeStruct((_N_PAD,), jnp.float32)
    return pl.pallas_call(
        _point_stage_body,
        grid=(grid,),
        in_specs=[
            pl.BlockSpec((_BLK, 4), lambda i: (i, 0)),
            pl.BlockSpec((10, 64), lambda i: (0, 0)),
            pl.BlockSpec((64,), lambda i: (0,)),
        ],
        out_specs=[
            pl.BlockSpec((_BLK,), lambda i: (i,)),
            pl.BlockSpec((_BLK,), lambda i: (i,)),
            pl.BlockSpec((_BLK,), lambda i: (i,)),
            pl.BlockSpec((_BLK,), lambda i: (i,)),
            pl.BlockSpec((_BLK, 128), lambda i: (i, 0)),
        ],
        out_shape=[
            jax.ShapeDtypeStruct((_N_PAD,), jnp.int32),
            out1d, out1d, out1d,
            jax.ShapeDtypeStruct((_N_PAD, 128), jnp.float32),
        ],
    )(points_pad, w, gamma)


_PBLK = 1024
_TB = _NSEG_PAD // _PBLK


def _pillar_stage_body(cnt0_ref, cnt1_ref, sx0_ref, sx1_ref, sy0_ref,
                       sy1_ref, sz0_ref, sz1_ref, m_ref, w_ref, gamma_ref,
                       beta_ref, out_ref):
    i = pl.program_id(0)
    count = (cnt0_ref[...] + cnt1_ref[...])[:, None]
    den = jnp.maximum(count, 1.0)
    mx = (sx0_ref[...] + sx1_ref[...])[:, None] / den
    my = (sy0_ref[...] + sy1_ref[...])[:, None] / den
    mz = (sz0_ref[...] + sz1_ref[...])[:, None] / den
    p = i * _PBLK + jax.lax.broadcasted_iota(jnp.int32, (_PBLK, 1), 0)
    ix = jax.lax.rem(p, _NX)
    iy = jax.lax.rem(p // _NX, _NY)
    cx = (ix.astype(jnp.float32) + 0.5) * _PILLAR
    cy = (iy.astype(jnp.float32) + 0.5) * _PILLAR
    wf = w_ref[...] * gamma_ref[...][None, :]
    v6 = jnp.concatenate(
        [-mx, -my, -mz, -cx, -cy,
         jnp.full((_PBLK, 1), -_ZC, jnp.float32)], axis=1)
    vw = v6[:, 0:1] * wf[4][None, :]
    for k in range(1, 6):
        vw = vw + v6[:, k:k + 1] * wf[4 + k][None, :]
    z = m_ref[...] + vw + beta_ref[...][None, :]
    out_ref[...] = jnp.where(count > 0, jnp.maximum(z, 0.0), 0.0)


def _pillar_stage(tab, m, w, gamma, beta):
    grid = _NSEG_PAD // _PBLK
    plane = [
        pl.BlockSpec((_PBLK,), lambda i, kk=k, cc=c: (kk * 4 * _TB
                                                      + cc * _TB + i,))
        for c in range(4) for k in range(2)
    ]
    return pl.pallas_call(
        _pillar_stage_body,
        grid=(grid,),
        in_specs=plane + [
            pl.BlockSpec((_PBLK, 64), lambda i: (i, 0)),
            pl.BlockSpec((10, 64), lambda i: (0, 0)),
            pl.BlockSpec((64,), lambda i: (0,)),
            pl.BlockSpec((64,), lambda i: (0,)),
        ],
        out_specs=pl.BlockSpec((_PBLK, 64), lambda i: (i, 0)),
        out_shape=jax.ShapeDtypeStruct((_NSEG_PAD, 64), jnp.float32),
    )(*([tab] * 8), m, w, gamma, beta)


def kernel(points, xyz_batch_cnt, W, gamma, beta):
    del xyz_batch_cnt
    points_pad = jnp.pad(points, ((0, _N_PAD - _N), (0, 0)))
    pid, rx, ry, rz, s = _point_stage(points_pad, W, gamma)
    ones_v = jnp.where(pid < _NSEG, 1.0, 0.0)
    cnt = jax.ops.segment_sum(ones_v, pid, num_segments=_NSEG_PAD)
    sx = jax.ops.segment_sum(rx * ones_v, pid, num_segments=_NSEG_PAD)
    sy = jax.ops.segment_sum(ry * ones_v, pid, num_segments=_NSEG_PAD)
    sz = jax.ops.segment_sum(rz * ones_v, pid, num_segments=_NSEG_PAD)
    zed = jnp.zeros((4 * _NSEG_PAD,), jnp.float32)
    tab = jnp.concatenate([cnt, sx, sy, sz, zed])
    m = jax.ops.segment_max(s[:, :64], pid, num_segments=_NSEG_PAD)
    out = _pillar_stage(tab, m, W, gamma, beta)
    return out[:_NSEG]


# speedup vs baseline: 1.3328x; 1.3328x over previous
import jax
import jax.numpy as jnp
from jax.experimental import pallas as pl

_PC_MIN = (0.0, -40.0, -3.0)
_PILLAR = 0.32
_NX = 220
_NY = 250
_NSEG = _NX * _NY
_NSEG_PAD = 55040
_N = 200000
_N_PAD = 204800
_BLK = 4096
_ZC = 2.0


def _point_stage_body(pts_ref, w_ref, gamma_ref, pid_ref, val4_ref, s_ref):
    i = pl.program_id(0)
    pts = pts_ref[...]
    relx = pts[:, 0:1] - _PC_MIN[0]
    rely = pts[:, 1:2] - _PC_MIN[1]
    relz = pts[:, 2:3] - _PC_MIN[2]
    ix = jnp.clip(jnp.floor(relx / _PILLAR).astype(jnp.int32), 0, _NX - 1)
    iy = jnp.clip(jnp.floor(rely / _PILLAR).astype(jnp.int32), 0, _NY - 1)
    pid = iy * _NX + ix
    gidx = i * _BLK + jax.lax.broadcasted_iota(jnp.int32, (_BLK, 1), 0)
    valid = gidx < _N
    pid = jnp.where(valid, pid, _NSEG_PAD - 1)
    pid_ref[...] = pid[:, 0]
    ones = jnp.where(valid, 1.0, 0.0).astype(jnp.float32)
    rel = jnp.concatenate([relx, rely, relz], axis=1)
    relm = jnp.where(valid, rel, 0.0)
    val4_ref[...] = jnp.concatenate([ones, relm], axis=1)
    wf = w_ref[...] * gamma_ref[...][None, :]
    u = jnp.concatenate([pts, rel, rel], axis=1)
    s = u[:, 0:1] * wf[0][None, :]
    for k in range(1, 10):
        s = s + u[:, k:k + 1] * wf[k][None, :]
    s_ref[...] = s


def _point_stage(points_pad, w, gamma):
    grid = _N_PAD // _BLK
    return pl.pallas_call(
        _point_stage_body,
        grid=(grid,),
        in_specs=[
            pl.BlockSpec((_BLK, 4), lambda i: (i, 0)),
            pl.BlockSpec((10, 64), lambda i: (0, 0)),
            pl.BlockSpec((64,), lambda i: (0,)),
        ],
        out_specs=[
            pl.BlockSpec((_BLK,), lambda i: (i,)),
            pl.BlockSpec((_BLK, 4), lambda i: (i, 0)),
            pl.BlockSpec((_BLK, 64), lambda i: (i, 0)),
        ],
        out_shape=[
            jax.ShapeDtypeStruct((_N_PAD,), jnp.int32),
            jax.ShapeDtypeStruct((_N_PAD, 4), jnp.float32),
            jax.ShapeDtypeStruct((_N_PAD, 64), jnp.float32),
        ],
    )(points_pad, w, gamma)


_PBLK = 1280


def _pillar_stage_body(tab_ref, m_ref, w_ref, gamma_ref, beta_ref, out_ref):
    i = pl.program_id(0)
    tab = tab_ref[0]
    count = tab[:, 0:1]
    mean = tab[:, 1:4] / jnp.maximum(count, 1.0)
    p = i * _PBLK + jax.lax.broadcasted_iota(jnp.int32, (_PBLK, 1), 0)
    ix = jax.lax.rem(p, _NX)
    iy = jax.lax.rem(p // _NX, _NY)
    cx = (ix.astype(jnp.float32) + 0.5) * _PILLAR
    cy = (iy.astype(jnp.float32) + 0.5) * _PILLAR
    wf = w_ref[...] * gamma_ref[...][None, :]
    v6 = jnp.concatenate(
        [-mean, -cx, -cy, jnp.full((_PBLK, 1), -_ZC, jnp.float32)], axis=1)
    vw = v6[:, 0:1] * wf[4][None, :]
    for k in range(1, 6):
        vw = vw + v6[:, k:k + 1] * wf[4 + k][None, :]
    z = m_ref[...] + vw + beta_ref[...][None, :]
    out_ref[...] = jnp.where(count > 0, jnp.maximum(z, 0.0), 0.0)


def _pillar_stage(tab2, m, w, gamma, beta):
    grid = _NSEG_PAD // _PBLK
    return pl.pallas_call(
        _pillar_stage_body,
        grid=(grid,),
        in_specs=[
            pl.BlockSpec((1, _PBLK, 4), lambda i: (0, i, 0)),
            pl.BlockSpec((_PBLK, 64), lambda i: (i, 0)),
            pl.BlockSpec((10, 64), lambda i: (0, 0)),
            pl.BlockSpec((64,), lambda i: (0,)),
            pl.BlockSpec((64,), lambda i: (0,)),
        ],
        out_specs=pl.BlockSpec((_PBLK, 64), lambda i: (i, 0)),
        out_shape=jax.ShapeDtypeStruct((_NSEG_PAD, 64), jnp.float32),
    )(tab2, m, w, gamma, beta)


def kernel(points, xyz_batch_cnt, W, gamma, beta):
    del xyz_batch_cnt
    points_pad = jnp.pad(points, ((0, _N_PAD - _N), (0, 0)))
    pid, val4, s = _point_stage(points_pad, W, gamma)
    tab = jax.ops.segment_sum(val4, pid, num_segments=_NSEG_PAD)
    tab2 = tab[None]
    m = jax.ops.segment_max(s, pid, num_segments=_NSEG_PAD)
    out = _pillar_stage(tab2, m, W, gamma, beta)
    return out[:_NSEG]
